# baseline (speedup 1.0000x reference)
import functools

import jax
import jax.numpy as jnp
from jax import lax
from jax.experimental import pallas as pl
from jax.experimental.pallas import tpu as pltpu
from jax.experimental.pallas import tpu_sc as plsc

N = 10000
E = 320000
D = 128
NC, NS, L = 2, 16, 16
NW = NC * NS
EPW = E // NW
CHUNK = 80
NCHUNK = EPW // CHUNK
NP = 10240
RPS = NP // NS
WRPS = 632
WRPL = N - (NS - 1) * WRPS
SLOPE = 0.01

_SC_PARAMS = pltpu.CompilerParams(needs_layout_passes=False)
_MESH = plsc.VectorSubcoreMesh(
    core_axis_name="c", subcore_axis_name="s", num_cores=NC, num_subcores=NS)


def _proj_body(x_ref, w_ref, a12_ref, h_ref, a_ref):
    h = jnp.dot(x_ref[...], w_ref[...], preferred_element_type=jnp.float32)
    h_ref[...] = h
    a_ref[...] = jnp.dot(h, a12_ref[...], preferred_element_type=jnp.float32)


def _scores_body(send_hbm, recv_hbm, a1_hbm, a2_hbm,
                 ex_hbm, den_hbm, cnt_hbm,
                 send_v, recv_v, ex_v, a1_v, a2_v, den_v, cnt_v):
    cid = lax.axis_index("c")
    sid = lax.axis_index("s")
    wid = cid * NS + sid

    pltpu.sync_copy(send_hbm.at[wid], send_v)
    pltpu.sync_copy(recv_hbm.at[wid], recv_v)
    pltpu.sync_copy(a1_hbm, a1_v)
    pltpu.sync_copy(a2_hbm, a2_v)

    def _zero(i, _):
        den_v[pl.ds(i * L, L)] = jnp.zeros((L,), jnp.float32)
        cnt_v[pl.ds(i * L, L)] = jnp.zeros((L,), jnp.float32)
        return 0

    lax.fori_loop(0, NP // L, _zero, 0)

    ones_f = jnp.ones((L,), jnp.float32)

    def _score(c, _):
        for j in range(CHUNK // L):
            s_idx = send_v[c, pl.ds(j * L, L)]
            r_idx = recv_v[c, pl.ds(j * L, L)]
            a1s = plsc.load_gather(a1_v, [s_idx])
            a2r = plsc.load_gather(a2_v, [r_idx])
            sc = a1s + a2r
            sc = jnp.where(sc > 0.0, sc, sc * SLOPE)
            e = jnp.exp(sc)
            ex_v[c, pl.ds(j * L, L)] = e
            plsc.addupdate_scatter(den_v, [r_idx], e)
            plsc.addupdate_scatter(cnt_v, [r_idx], ones_f)
        return 0

    lax.fori_loop(0, NCHUNK, _score, 0)

    pltpu.sync_copy(ex_v, ex_hbm.at[wid])
    pltpu.sync_copy(den_v, den_hbm.at[wid])
    pltpu.sync_copy(cnt_v, cnt_hbm.at[wid])


def _agg_body(send_hbm, recv_hbm, ex_hbm, h_hbm, acc_hbm,
              send_v, rc_v, exc_v, rows_v, acc_sh, rsem, esem, gsem, ssem):
    cid = lax.axis_index("c")
    sid = lax.axis_index("s")
    wid = cid * NS + sid

    def _zero_rows(i, _):
        for j in range(D // L):
            rows_v[0, i, pl.ds(j * L, L)] = jnp.zeros((L,), jnp.float32)
        return 0

    lax.fori_loop(0, CHUNK, _zero_rows, 0)
    base = sid * WRPS

    @pl.when(sid < NS - 1)
    def _():
        for k in range(WRPS // CHUNK):
            pltpu.sync_copy(rows_v.at[0],
                            acc_sh.at[pl.ds(base + k * CHUNK, CHUNK)])
        pltpu.sync_copy(rows_v.at[0, pl.ds(0, WRPS % CHUNK)],
                        acc_sh.at[pl.ds(base + WRPS - WRPS % CHUNK,
                                        WRPS % CHUNK)])

    @pl.when(sid == NS - 1)
    def _():
        for k in range(WRPL // CHUNK):
            pltpu.sync_copy(rows_v.at[0],
                            acc_sh.at[pl.ds(base + k * CHUNK, CHUNK)])
        pltpu.sync_copy(rows_v.at[0, pl.ds(0, WRPL % CHUNK)],
                        acc_sh.at[pl.ds(base + WRPL - WRPL % CHUNK,
                                        WRPL % CHUNK)])

    plsc.subcore_barrier()

    pltpu.sync_copy(send_hbm.at[wid], send_v)

    def _stage(c, b):
        pltpu.async_copy(recv_hbm.at[wid, c], rc_v.at[b], rsem.at[b])
        pltpu.async_copy(ex_hbm.at[wid, c], exc_v.at[b], esem.at[b])

    def _stage_wait(c, b):
        pltpu.make_async_copy(recv_hbm.at[wid, c], rc_v.at[b],
                              rsem.at[b]).wait()
        pltpu.make_async_copy(ex_hbm.at[wid, c], exc_v.at[b],
                              esem.at[b]).wait()

    def _gather(c, b):
        pltpu.async_copy(h_hbm.at[send_v.at[c]], rows_v.at[b], gsem.at[b])

    def _gather_wait(c, b):
        pltpu.make_async_copy(h_hbm.at[send_v.at[c]], rows_v.at[b],
                              gsem.at[b]).wait()

    def _scatter(c, b):
        pltpu.async_copy(rows_v.at[b], acc_sh.at[rc_v.at[b]],
                         ssem.at[b], add=True)

    def _scatter_wait(b):
        pltpu.make_async_copy(rows_v.at[b], acc_sh.at[rc_v.at[b]],
                              ssem.at[b]).wait()

    _stage(0, 0)
    _gather(0, 0)

    def _chunk(c, _):
        b = lax.rem(c, 2)
        nb = 1 - b

        @pl.when(c + 1 < NCHUNK)
        def _():
            @pl.when(c >= 1)
            def _():
                _scatter_wait(nb)
            _stage(c + 1, nb)
            _gather(c + 1, nb)

        _gather_wait(c, b)
        _stage_wait(c, b)

        def _scale(g, _):
            ev = exc_v[b, pl.ds(g * L, L)]
            rb = g * L
            for i in range(L):
                cval = ev[i]
                for j in range(D // L):
                    rows_v[b, rb + i, pl.ds(j * L, L)] = (
                        rows_v[b, rb + i, pl.ds(j * L, L)] * cval)
            return 0

        lax.fori_loop(0, CHUNK // L, _scale, 0)
        _scatter(c, b)
        return 0

    lax.fori_loop(0, NCHUNK, _chunk, 0)
    _scatter_wait((NCHUNK - 1) % 2)
    _scatter_wait(NCHUNK % 2)
    plsc.subcore_barrier()

    @pl.when(sid < NS - 1)
    def _():
        pltpu.sync_copy(acc_sh.at[pl.ds(base, WRPS)],
                        acc_hbm.at[cid, pl.ds(base, WRPS)])

    @pl.when(sid == NS - 1)
    def _():
        pltpu.sync_copy(acc_sh.at[pl.ds(base, WRPL)],
                        acc_hbm.at[cid, pl.ds(base, WRPL)])


def _final_body(acc_ref, den_ref, cnt_ref, out_ref):
    s = acc_ref[0] + acc_ref[1]
    den = jnp.sum(den_ref[...], axis=0)[:N]
    cnt = jnp.sum(cnt_ref[...], axis=0)[:N]
    scale = 1.0 / (jnp.maximum(den, 1e-30) * jnp.maximum(cnt, 1.0))
    out_ref[...] = s * scale[:, None]


def kernel(x, edge_index, W, A):
    senders = edge_index[0].reshape(NW, NCHUNK, CHUNK)
    receivers = edge_index[1].reshape(NW, NCHUNK, CHUNK)
    a12 = A.reshape(2, D).T

    h, a = pl.pallas_call(
        _proj_body,
        out_shape=[
            jax.ShapeDtypeStruct((N, D), jnp.float32),
            jax.ShapeDtypeStruct((N, 2), jnp.float32),
        ],
    )(x, W, a12)

    ex, den, cnt = pl.kernel(
        _scores_body,
        out_type=[
            jax.ShapeDtypeStruct((NW, NCHUNK, CHUNK), jnp.float32),
            jax.ShapeDtypeStruct((NW, NP), jnp.float32),
            jax.ShapeDtypeStruct((NW, NP), jnp.float32),
        ],
        mesh=_MESH,
        compiler_params=_SC_PARAMS,
        scratch_types=[
            pltpu.VMEM((NCHUNK, CHUNK), jnp.int32),
            pltpu.VMEM((NCHUNK, CHUNK), jnp.int32),
            pltpu.VMEM((NCHUNK, CHUNK), jnp.float32),
            pltpu.VMEM((N,), jnp.float32),
            pltpu.VMEM((N,), jnp.float32),
            pltpu.VMEM((NP,), jnp.float32),
            pltpu.VMEM((NP,), jnp.float32),
        ],
    )(senders, receivers, a[:, 0], a[:, 1])

    acc = pl.kernel(
        _agg_body,
        out_type=jax.ShapeDtypeStruct((NC, N, D), jnp.float32),
        mesh=_MESH,
        compiler_params=_SC_PARAMS,
        scratch_types=[
            pltpu.VMEM((NCHUNK, CHUNK), jnp.int32),
            pltpu.VMEM((2, CHUNK), jnp.int32),
            pltpu.VMEM((2, CHUNK), jnp.float32),
            pltpu.VMEM((2, CHUNK, D), jnp.float32),
            pltpu.VMEM_SHARED((N, D), jnp.float32),
            pltpu.SemaphoreType.DMA((2,)),
            pltpu.SemaphoreType.DMA((2,)),
            pltpu.SemaphoreType.DMA((2,)),
            pltpu.SemaphoreType.DMA((2,)),
        ],
    )(senders, receivers, ex, h)

    out = pl.pallas_call(
        _final_body,
        out_shape=jax.ShapeDtypeStruct((N, D), jnp.float32),
    )(acc, den, cnt)
    return out


# speedup vs baseline: 2.2475x; 2.2475x over previous
import functools

import jax
import jax.numpy as jnp
from jax import lax
from jax.experimental import pallas as pl
from jax.experimental.pallas import tpu as pltpu
from jax.experimental.pallas import tpu_sc as plsc

N = 10000
E = 320000
D = 128
NC, NS, L = 2, 16, 16
NW = NC * NS
EPW = E // NW
CHUNK = 80
NCHUNK = EPW // CHUNK
NP = 10240
RPS = NP // NS
WRPS = 632
WRPL = N - (NS - 1) * WRPS
SLOPE = 0.01

_SC_PARAMS = pltpu.CompilerParams(needs_layout_passes=False)
_MESH = plsc.VectorSubcoreMesh(
    core_axis_name="c", subcore_axis_name="s", num_cores=NC, num_subcores=NS)


def _proj_body(x_ref, w_ref, a12_ref, h_ref, a_ref):
    h = jnp.dot(x_ref[...], w_ref[...], preferred_element_type=jnp.float32)
    h_ref[...] = h
    a_ref[...] = jnp.dot(h, a12_ref[...], preferred_element_type=jnp.float32)


def _scores_body(send_hbm, recv_hbm, a1_hbm, a2_hbm,
                 ex_hbm, den_hbm, cnt_hbm,
                 send_v, recv_v, ex_v, a1_v, a2_v, den_v, cnt_v):
    cid = lax.axis_index("c")
    sid = lax.axis_index("s")
    wid = cid * NS + sid

    pltpu.sync_copy(send_hbm.at[wid], send_v)
    pltpu.sync_copy(recv_hbm.at[wid], recv_v)
    pltpu.sync_copy(a1_hbm, a1_v)
    pltpu.sync_copy(a2_hbm, a2_v)

    def _zero(i, _):
        den_v[pl.ds(i * L, L)] = jnp.zeros((L,), jnp.float32)
        cnt_v[pl.ds(i * L, L)] = jnp.zeros((L,), jnp.float32)
        return 0

    lax.fori_loop(0, NP // L, _zero, 0)

    ones_f = jnp.ones((L,), jnp.float32)

    def _score(c, _):
        for j in range(CHUNK // L):
            s_idx = send_v[c, pl.ds(j * L, L)]
            r_idx = recv_v[c, pl.ds(j * L, L)]
            a1s = plsc.load_gather(a1_v, [s_idx])
            a2r = plsc.load_gather(a2_v, [r_idx])
            sc = a1s + a2r
            sc = jnp.where(sc > 0.0, sc, sc * SLOPE)
            e = jnp.exp(sc)
            ex_v[c, pl.ds(j * L, L)] = e
            plsc.addupdate_scatter(den_v, [r_idx], e)
            plsc.addupdate_scatter(cnt_v, [r_idx], ones_f)
        return 0

    lax.fori_loop(0, NCHUNK, _score, 0)

    pltpu.sync_copy(ex_v, ex_hbm.at[wid])
    pltpu.sync_copy(den_v, den_hbm.at[wid])
    pltpu.sync_copy(cnt_v, cnt_hbm.at[wid])


def _agg_body(send_hbm, recv_hbm, ex_hbm, h_hbm, acc_hbm,
              send_v, rc_v, exc_v, rows_v, acc_sh, rsem, esem, gsem, ssem):
    cid = lax.axis_index("c")
    sid = lax.axis_index("s")
    wid = cid * NS + sid

    def _zero_rows(i, _):
        for j in range(D // L):
            rows_v[0, i, pl.ds(j * L, L)] = jnp.zeros((L,), jnp.float32)
        return 0

    lax.fori_loop(0, CHUNK, _zero_rows, 0)
    base = sid * WRPS

    @pl.when(sid < NS - 1)
    def _():
        for k in range(WRPS // CHUNK):
            pltpu.sync_copy(rows_v.at[0],
                            acc_sh.at[pl.ds(base + k * CHUNK, CHUNK)])
        pltpu.sync_copy(rows_v.at[0, pl.ds(0, WRPS % CHUNK)],
                        acc_sh.at[pl.ds(base + WRPS - WRPS % CHUNK,
                                        WRPS % CHUNK)])

    @pl.when(sid == NS - 1)
    def _():
        for k in range(WRPL // CHUNK):
            pltpu.sync_copy(rows_v.at[0],
                            acc_sh.at[pl.ds(base + k * CHUNK, CHUNK)])
        pltpu.sync_copy(rows_v.at[0, pl.ds(0, WRPL % CHUNK)],
                        acc_sh.at[pl.ds(base + WRPL - WRPL % CHUNK,
                                        WRPL % CHUNK)])

    plsc.subcore_barrier()

    pltpu.sync_copy(send_hbm.at[wid], send_v)

    def _stage(c, b):
        pltpu.async_copy(recv_hbm.at[wid, c], rc_v.at[b], rsem.at[b])
        pltpu.async_copy(ex_hbm.at[wid, c], exc_v.at[b], esem.at[b])

    def _stage_wait(c, b):
        pltpu.make_async_copy(recv_hbm.at[wid, c], rc_v.at[b],
                              rsem.at[b]).wait()
        pltpu.make_async_copy(ex_hbm.at[wid, c], exc_v.at[b],
                              esem.at[b]).wait()

    def _gather(c, b):
        pltpu.async_copy(h_hbm.at[send_v.at[c]], rows_v.at[b], gsem.at[b])

    def _gather_wait(c, b):
        pltpu.make_async_copy(h_hbm.at[send_v.at[c]], rows_v.at[b],
                              gsem.at[b]).wait()

    def _scatter(c, b):
        pltpu.async_copy(rows_v.at[b], acc_sh.at[rc_v.at[b]],
                         ssem.at[b], add=True)

    def _scatter_wait(b):
        pltpu.make_async_copy(rows_v.at[b], acc_sh.at[rc_v.at[b]],
                              ssem.at[b]).wait()

    _stage(0, 0)
    _gather(0, 0)

    def _chunk(c, _):
        b = lax.rem(c, 2)
        nb = 1 - b

        @pl.when(c + 1 < NCHUNK)
        def _():
            @pl.when(c >= 1)
            def _():
                _scatter_wait(nb)
            _stage(c + 1, nb)
            _gather(c + 1, nb)

        _gather_wait(c, b)
        _stage_wait(c, b)

        @plsc.parallel_loop(0, CHUNK // L, unroll=2)
        def _scale(g):
            ev = exc_v[b, pl.ds(g * L, L)]
            rb = g * L
            for i in range(L):
                cval = ev[i]
                vals = [rows_v[b, rb + i, pl.ds(j * L, L)] * cval
                        for j in range(D // L)]
                for j in range(D // L):
                    rows_v[b, rb + i, pl.ds(j * L, L)] = vals[j]

        _scatter(c, b)
        return 0

    lax.fori_loop(0, NCHUNK, _chunk, 0)
    _scatter_wait((NCHUNK - 1) % 2)
    _scatter_wait(NCHUNK % 2)
    plsc.subcore_barrier()

    @pl.when(sid < NS - 1)
    def _():
        pltpu.sync_copy(acc_sh.at[pl.ds(base, WRPS)],
                        acc_hbm.at[cid, pl.ds(base, WRPS)])

    @pl.when(sid == NS - 1)
    def _():
        pltpu.sync_copy(acc_sh.at[pl.ds(base, WRPL)],
                        acc_hbm.at[cid, pl.ds(base, WRPL)])


def _final_body(acc_ref, den_ref, cnt_ref, out_ref):
    s = acc_ref[0] + acc_ref[1]
    den = jnp.sum(den_ref[...], axis=0)[:N]
    cnt = jnp.sum(cnt_ref[...], axis=0)[:N]
    scale = 1.0 / (jnp.maximum(den, 1e-30) * jnp.maximum(cnt, 1.0))
    out_ref[...] = s * scale[:, None]


def kernel(x, edge_index, W, A):
    senders = edge_index[0].reshape(NW, NCHUNK, CHUNK)
    receivers = edge_index[1].reshape(NW, NCHUNK, CHUNK)
    a12 = A.reshape(2, D).T

    h, a = pl.pallas_call(
        _proj_body,
        out_shape=[
            jax.ShapeDtypeStruct((N, D), jnp.float32),
            jax.ShapeDtypeStruct((N, 2), jnp.float32),
        ],
    )(x, W, a12)

    ex, den, cnt = pl.kernel(
        _scores_body,
        out_type=[
            jax.ShapeDtypeStruct((NW, NCHUNK, CHUNK), jnp.float32),
            jax.ShapeDtypeStruct((NW, NP), jnp.float32),
            jax.ShapeDtypeStruct((NW, NP), jnp.float32),
        ],
        mesh=_MESH,
        compiler_params=_SC_PARAMS,
        scratch_types=[
            pltpu.VMEM((NCHUNK, CHUNK), jnp.int32),
            pltpu.VMEM((NCHUNK, CHUNK), jnp.int32),
            pltpu.VMEM((NCHUNK, CHUNK), jnp.float32),
            pltpu.VMEM((N,), jnp.float32),
            pltpu.VMEM((N,), jnp.float32),
            pltpu.VMEM((NP,), jnp.float32),
            pltpu.VMEM((NP,), jnp.float32),
        ],
    )(senders, receivers, a[:, 0], a[:, 1])

    acc = pl.kernel(
        _agg_body,
        out_type=jax.ShapeDtypeStruct((NC, N, D), jnp.float32),
        mesh=_MESH,
        compiler_params=_SC_PARAMS,
        scratch_types=[
            pltpu.VMEM((NCHUNK, CHUNK), jnp.int32),
            pltpu.VMEM((2, CHUNK), jnp.int32),
            pltpu.VMEM((2, CHUNK), jnp.float32),
            pltpu.VMEM((2, CHUNK, D), jnp.float32),
            pltpu.VMEM_SHARED((N, D), jnp.float32),
            pltpu.SemaphoreType.DMA((2,)),
            pltpu.SemaphoreType.DMA((2,)),
            pltpu.SemaphoreType.DMA((2,)),
            pltpu.SemaphoreType.DMA((2,)),
        ],
    )(senders, receivers, ex, h)

    out = pl.pallas_call(
        _final_body,
        out_shape=jax.ShapeDtypeStruct((N, D), jnp.float32),
    )(acc, den, cnt)
    return out
